# baseline (speedup 1.0000x reference)
import jax
import jax.numpy as jnp
from jax import lax
from jax.experimental import pallas as pl
from jax.experimental.pallas import tpu as pltpu
from jax.experimental.pallas import tpu_sc as plsc

F = 26
V = 100000
D = 32
B = 16384

_info = plsc.get_sparse_core_info()
NC = _info.num_cores
NSUB = _info.num_subcores
L = _info.num_lanes
NW = NC * NSUB
H0 = 50048
H1 = V - H0
IC = 4096
NCHK = B // IC
UNROLL = 8


def _body(idx_hbm, tab_hbm, out_hbm,
          row_a, row_b, idx0, idx1, acc,
          sem_a, sem_b, sem_i0, sem_i1):
    c = lax.axis_index("c")
    s = lax.axis_index("s")
    d = c * NSUB + s

    ibufs = (idx0, idx1)
    isems = (sem_i0, sem_i1)

    def _zero(j, _z):
        for u8 in range(UNROLL):
            acc[pl.ds((j * UNROLL + u8) * L, L)] = jnp.zeros((L,), jnp.float32)
        return 0

    lax.fori_loop(0, B // (L * UNROLL), _zero, 0)

    def _pass(i, buf, lo, sz):
        pltpu.async_copy(idx_hbm.at[i, pl.ds(0, IC)], ibufs[0], isems[0])
        for ch in range(NCHK):
            ib, sm = ibufs[ch % 2], isems[ch % 2]
            pltpu.make_async_copy(idx_hbm.at[0, pl.ds(0, IC)], ib, sm).wait()
            if ch + 1 < NCHK:
                pltpu.async_copy(idx_hbm.at[i, pl.ds((ch + 1) * IC, IC)],
                                 ibufs[(ch + 1) % 2], isems[(ch + 1) % 2])
            base = ch * IC

            def _g(j, _g2):
                for u8 in range(UNROLL):
                    off = (j * UNROLL + u8) * L
                    v = ib[pl.ds(off, L)]
                    u = v - lo if lo else v
                    m = u.astype(jnp.uint32) < jnp.uint32(sz)
                    vals = plsc.load_gather(buf, [u], mask=m)
                    vals = jnp.where(m, vals, 0.0)
                    plsc.addupdate(acc.at[pl.ds(base + off, L)], vals)
                return 0

            lax.fori_loop(0, IC // (L * UNROLL), _g, 0)

    pltpu.async_copy(tab_hbm.at[0, d, pl.ds(0, H0)], row_a, sem_a)

    def _field(i, _f):
        pltpu.make_async_copy(tab_hbm.at[0, 0, pl.ds(0, H0)], row_a, sem_a).wait()
        pltpu.async_copy(tab_hbm.at[i, d, pl.ds(H0, H1)], row_b, sem_b)
        _pass(i, row_a, 0, H0)

        pltpu.make_async_copy(tab_hbm.at[0, 0, pl.ds(H0, H1)], row_b, sem_b).wait()
        inext = jnp.minimum(i + 1, F - 1)

        @pl.when(i + 1 < F)
        def _prefetch_next():
            pltpu.async_copy(tab_hbm.at[inext, d, pl.ds(0, H0)], row_a, sem_a)

        _pass(i, row_b, H0, H1)
        return 0

    lax.fori_loop(0, F, _field, 0)
    pltpu.sync_copy(acc, out_hbm.at[d])


def kernel(inputs, tables):
    idx_t = jnp.transpose(inputs).astype(jnp.int32)
    tab_t = jnp.transpose(tables, (0, 2, 1))
    mesh = plsc.VectorSubcoreMesh(core_axis_name="c", subcore_axis_name="s")
    f = pl.kernel(
        _body,
        out_type=jax.ShapeDtypeStruct((D, B), jnp.float32),
        mesh=mesh,
        scratch_types=[
            pltpu.VMEM((H0,), jnp.float32),
            pltpu.VMEM((H1,), jnp.float32),
            pltpu.VMEM((IC,), jnp.int32),
            pltpu.VMEM((IC,), jnp.int32),
            pltpu.VMEM((B,), jnp.float32),
            pltpu.SemaphoreType.DMA,
            pltpu.SemaphoreType.DMA,
            pltpu.SemaphoreType.DMA,
            pltpu.SemaphoreType.DMA,
        ],
        compiler_params=pltpu.CompilerParams(needs_layout_passes=False),
    )
    out_t = f(idx_t, tab_t)
    return jnp.transpose(out_t)


# speedup vs baseline: 1.5066x; 1.5066x over previous
import jax
import jax.numpy as jnp
from jax import lax
from jax.experimental import pallas as pl
from jax.experimental.pallas import tpu as pltpu
from jax.experimental.pallas import tpu_sc as plsc

F = 26
V = 100000
D = 32
B = 16384

_info = plsc.get_sparse_core_info()
NC = _info.num_cores
NSUB = _info.num_subcores
L = _info.num_lanes
NW = NC * NSUB
IC = 4096
NCHK = B // IC
UNROLL = 8


def _body(idx_hbm, tab_hbm, out_hbm,
          rowbuf, idx0, idx1, acc, sem_i0, sem_i1):
    c = lax.axis_index("c")
    s = lax.axis_index("s")
    d = c * NSUB + s

    ibufs = (idx0, idx1)
    isems = (sem_i0, sem_i1)

    @plsc.parallel_loop(0, B, step=L)
    def _zero(b):
        acc[pl.ds(b, L)] = jnp.zeros((L,), jnp.float32)

    pltpu.async_copy(idx_hbm.at[0, pl.ds(0, IC)], ibufs[0], isems[0])

    def _field(i, _f):
        pltpu.sync_copy(tab_hbm.at[i, d], rowbuf)
        inext = jnp.minimum(i + 1, F - 1)
        for ch in range(NCHK):
            p = ch % 2
            q = (ch + 1) % 2
            ib = ibufs[p]
            pltpu.make_async_copy(idx_hbm.at[0, pl.ds(0, IC)], ib, isems[p]).wait()
            if ch + 1 < NCHK:
                pltpu.async_copy(idx_hbm.at[i, pl.ds((ch + 1) * IC, IC)],
                                 ibufs[q], isems[q])
            else:
                @pl.when(i + 1 < F)
                def _prefetch_next_field():
                    pltpu.async_copy(idx_hbm.at[inext, pl.ds(0, IC)],
                                     ibufs[q], isems[q])
            base = ch * IC

            @plsc.parallel_loop(0, IC, step=L, unroll=UNROLL)
            def _gather(b):
                v = ib[pl.ds(b, L)]
                vals = plsc.load_gather(rowbuf, [v])
                plsc.addupdate(acc.at[pl.ds(base + b, L)], vals)

        return 0

    lax.fori_loop(0, F, _field, 0)
    pltpu.sync_copy(acc, out_hbm.at[d])


def kernel(inputs, tables):
    idx_t = jnp.transpose(inputs).astype(jnp.int32)
    tab_t = jnp.transpose(tables, (0, 2, 1))
    mesh = plsc.VectorSubcoreMesh(core_axis_name="c", subcore_axis_name="s")
    f = pl.kernel(
        _body,
        out_type=jax.ShapeDtypeStruct((D, B), jnp.float32),
        mesh=mesh,
        scratch_types=[
            pltpu.VMEM((V,), jnp.float32),
            pltpu.VMEM((IC,), jnp.int32),
            pltpu.VMEM((IC,), jnp.int32),
            pltpu.VMEM((B,), jnp.float32),
            pltpu.SemaphoreType.DMA,
            pltpu.SemaphoreType.DMA,
        ],
        compiler_params=pltpu.CompilerParams(needs_layout_passes=False),
    )
    out_t = f(idx_t, tab_t)
    return jnp.transpose(out_t)
